# Initial kernel scaffold; baseline (speedup 1.0000x reference)
import jax
import jax.numpy as jnp
from jax.experimental import pallas as pl


def kernel(x_nchw, conv_weight, gamma, beta):
    raise NotImplementedError("write your pallas kernel here")


# speedup vs baseline: 3.0060x; 3.0060x over previous
import jax
import jax.numpy as jnp
from jax.experimental import pallas as pl
from jax.experimental.pallas import tpu as pltpu

_BN_EPS = 1e-5


def _gram_kernel(x_ref, g_ref, s_ref):
    x = x_ref[0]
    g_ref[0] = jax.lax.dot_general(
        x, x, (((1,), (1,)), ((), ())),
        preferred_element_type=jnp.float32)
    s_ref[0] = jnp.sum(x, axis=-1, keepdims=True)


def _apply_kernel(x_ref, w_ref, shift_ref, o_ref):
    x = x_ref[0]
    y = jnp.dot(w_ref[...], x, preferred_element_type=jnp.float32)
    o_ref[0] = (y + shift_ref[...]).astype(o_ref.dtype)


def kernel(x_nchw, conv_weight, gamma, beta):
    n, c_in, h, w_sp = x_nchw.shape
    c_out = conv_weight.shape[0]
    hw = h * w_sp
    m = n * hw
    itemsize = jnp.dtype(x_nchw.dtype).itemsize

    x3 = x_nchw.reshape(n, c_in, hw)
    w_mat = conv_weight[:, :, 0, 0].astype(jnp.float32)

    gp, sp = pl.pallas_call(
        _gram_kernel,
        out_shape=(
            jax.ShapeDtypeStruct((n, c_in, c_in), jnp.float32),
            jax.ShapeDtypeStruct((n, c_in, 1), jnp.float32),
        ),
        grid_spec=pltpu.PrefetchScalarGridSpec(
            num_scalar_prefetch=0,
            grid=(n,),
            in_specs=[
                pl.BlockSpec((1, c_in, hw), lambda i: (i, 0, 0)),
            ],
            out_specs=(
                pl.BlockSpec((1, c_in, c_in), lambda i: (i, 0, 0)),
                pl.BlockSpec((1, c_in, 1), lambda i: (i, 0, 0)),
            ),
        ),
        compiler_params=pltpu.CompilerParams(
            dimension_semantics=("parallel",)),
        cost_estimate=pl.CostEstimate(
            flops=2 * m * c_in * c_in + m * c_in,
            transcendentals=0,
            bytes_accessed=itemsize * m * c_in
            + 4 * n * (c_in * c_in + c_in)),
    )(x3)

    g = jnp.sum(gp, axis=0)
    s = jnp.sum(sp, axis=0)[:, 0]
    mean = (w_mat @ s) / m
    ey2 = jnp.sum((w_mat @ g) * w_mat, axis=1) / m
    var = jnp.maximum(ey2 - mean * mean, 0.0)
    inv_std = jax.lax.rsqrt(var + _BN_EPS)
    scale = gamma.astype(jnp.float32) * inv_std
    shift = (beta.astype(jnp.float32) - mean * scale).reshape(c_out, 1)
    w_scaled = w_mat * scale[:, None]

    tile = 8192 if hw % 8192 == 0 else hw
    num_t = hw // tile
    out3 = pl.pallas_call(
        _apply_kernel,
        out_shape=jax.ShapeDtypeStruct((n, c_out, hw), x_nchw.dtype),
        grid_spec=pltpu.PrefetchScalarGridSpec(
            num_scalar_prefetch=0,
            grid=(n, num_t),
            in_specs=[
                pl.BlockSpec((1, c_in, tile), lambda b, t: (b, 0, t)),
                pl.BlockSpec((c_out, c_in), lambda b, t: (0, 0)),
                pl.BlockSpec((c_out, 1), lambda b, t: (0, 0)),
            ],
            out_specs=pl.BlockSpec((1, c_out, tile), lambda b, t: (b, 0, t)),
        ),
        compiler_params=pltpu.CompilerParams(
            dimension_semantics=("parallel", "parallel")),
        cost_estimate=pl.CostEstimate(
            flops=2 * m * c_in * c_out + m * c_out,
            transcendentals=0,
            bytes_accessed=itemsize * (m * c_in + m * c_out)
            + 4 * c_in * c_out),
    )(x3, w_scaled, shift)

    return out3.reshape(n, c_out, h, w_sp)


# speedup vs baseline: 4.1235x; 1.3717x over previous
import jax
import jax.numpy as jnp
from jax.experimental import pallas as pl
from jax.experimental.pallas import tpu as pltpu

_BN_EPS = 1e-5


def _gram_kernel(x_ref, g_ref, s_ref):
    x = x_ref[...]
    g_ref[0] = jax.lax.dot_general(
        x, x, (((1,), (1,)), ((), ())),
        preferred_element_type=jnp.float32)
    s_ref[0] = jnp.sum(x, axis=-1, keepdims=True)


def _apply_kernel(x_ref, w_ref, shift_ref, o_ref):
    y = jnp.dot(w_ref[...], x_ref[...],
                preferred_element_type=jnp.float32)
    o_ref[...] = (y + shift_ref[...]).astype(o_ref.dtype)


def kernel(x_nchw, conv_weight, gamma, beta):
    n, c_in, h, w_sp = x_nchw.shape
    c_out = conv_weight.shape[0]
    hw = h * w_sp
    m = n * hw
    itemsize = jnp.dtype(x_nchw.dtype).itemsize

    x2 = x_nchw.reshape(n * c_in, hw)
    w_mat = conv_weight[:, :, 0, 0].astype(jnp.float32)

    b1 = max(1, 8 // c_in)
    rows1 = b1 * c_in
    g1 = n // b1
    gp, sp = pl.pallas_call(
        _gram_kernel,
        out_shape=(
            jax.ShapeDtypeStruct((g1, rows1, rows1), jnp.float32),
            jax.ShapeDtypeStruct((g1, rows1, 1), jnp.float32),
        ),
        grid_spec=pltpu.PrefetchScalarGridSpec(
            num_scalar_prefetch=0,
            grid=(g1,),
            in_specs=[
                pl.BlockSpec((rows1, hw), lambda i: (i, 0)),
            ],
            out_specs=(
                pl.BlockSpec((1, rows1, rows1), lambda i: (i, 0, 0)),
                pl.BlockSpec((1, rows1, 1), lambda i: (i, 0, 0)),
            ),
        ),
        compiler_params=pltpu.CompilerParams(
            dimension_semantics=("parallel",)),
        cost_estimate=pl.CostEstimate(
            flops=2 * m * c_in * rows1 + m * c_in,
            transcendentals=0,
            bytes_accessed=itemsize * m * c_in
            + 4 * g1 * (rows1 * rows1 + rows1)),
    )(x2)

    g_big = jnp.sum(gp, axis=0)
    s_big = jnp.sum(sp, axis=0)[:, 0]
    g = jnp.sum(g_big.reshape(b1, c_in, b1, c_in)
                [jnp.arange(b1), :, jnp.arange(b1), :], axis=0)
    s = jnp.sum(s_big.reshape(b1, c_in), axis=0)
    mean = (w_mat @ s) / m
    ey2 = jnp.sum((w_mat @ g) * w_mat, axis=1) / m
    var = jnp.maximum(ey2 - mean * mean, 0.0)
    inv_std = jax.lax.rsqrt(var + _BN_EPS)
    scale = gamma.astype(jnp.float32) * inv_std
    shift = beta.astype(jnp.float32) - mean * scale
    w_scaled = w_mat * scale[:, None]

    b2 = 4 if n % 4 == 0 else 1
    w_bd = jnp.kron(jnp.eye(b2, dtype=jnp.float32), w_scaled)
    shift_bd = jnp.tile(shift, b2).reshape(b2 * c_out, 1)
    tile = 8192 if hw % 8192 == 0 else hw
    num_t = hw // tile
    out2 = pl.pallas_call(
        _apply_kernel,
        out_shape=jax.ShapeDtypeStruct((n * c_out, hw), x_nchw.dtype),
        grid_spec=pltpu.PrefetchScalarGridSpec(
            num_scalar_prefetch=0,
            grid=(n // b2, num_t),
            in_specs=[
                pl.BlockSpec((b2 * c_in, tile), lambda b, t: (b, t)),
                pl.BlockSpec((b2 * c_out, b2 * c_in), lambda b, t: (0, 0)),
                pl.BlockSpec((b2 * c_out, 1), lambda b, t: (0, 0)),
            ],
            out_specs=pl.BlockSpec((b2 * c_out, tile), lambda b, t: (b, t)),
        ),
        compiler_params=pltpu.CompilerParams(
            dimension_semantics=("parallel", "parallel")),
        cost_estimate=pl.CostEstimate(
            flops=2 * m * c_in * c_out + m * c_out,
            transcendentals=0,
            bytes_accessed=itemsize * (m * c_in + m * c_out)
            + 4 * b2 * b2 * c_in * c_out),
    )(x2, w_bd, shift_bd)

    return out2.reshape(n, c_out, h, w_sp)


# speedup vs baseline: 9.7753x; 2.3706x over previous
import jax
import jax.numpy as jnp
from jax.experimental import pallas as pl
from jax.experimental.pallas import tpu as pltpu

_BN_EPS = 1e-5


def _make_stats_kernel(c_in):
    def _stats_kernel(x_ref, p_ref):
        xs = [x_ref[0, i] for i in range(c_in)]
        sums = {}
        for i in range(c_in):
            for j in range(i, c_in):
                sums[(i, j)] = jnp.sum(xs[i] * xs[j], axis=0)
        rows = [sums[(min(i, j), max(i, j))]
                for i in range(c_in) for j in range(c_in)]
        rows += [jnp.sum(xs[i], axis=0) for i in range(c_in)]
        p_ref[0] = jnp.stack(rows)
    return _stats_kernel


def _make_apply_kernel(c_in, c_out):
    def _apply_kernel(x_ref, w_ref, shift_ref, o_ref):
        xs = [x_ref[0, i] for i in range(c_in)]
        for o in range(c_out):
            acc = xs[0] * w_ref[o, 0] + shift_ref[o]
            for i in range(1, c_in):
                acc += xs[i] * w_ref[o, i]
            o_ref[0, o] = acc
    return _apply_kernel


def kernel(x_nchw, conv_weight, gamma, beta):
    n, c_in, h, w_sp = x_nchw.shape
    c_out = conv_weight.shape[0]
    m = n * h * w_sp
    itemsize = jnp.dtype(x_nchw.dtype).itemsize
    w_mat = conv_weight[:, :, 0, 0].astype(jnp.float32)
    nrows = c_in * c_in + c_in

    partials = pl.pallas_call(
        _make_stats_kernel(c_in),
        out_shape=jax.ShapeDtypeStruct((n, nrows, w_sp), jnp.float32),
        grid=(n,),
        in_specs=[
            pl.BlockSpec((1, c_in, h, w_sp), lambda i: (i, 0, 0, 0)),
        ],
        out_specs=pl.BlockSpec((1, nrows, w_sp), lambda i: (i, 0, 0)),
        compiler_params=pltpu.CompilerParams(
            dimension_semantics=("parallel",)),
        cost_estimate=pl.CostEstimate(
            flops=2 * m * (c_in * (c_in + 1) // 2 + c_in),
            transcendentals=0,
            bytes_accessed=itemsize * m * c_in + 4 * n * nrows * w_sp),
    )(x_nchw)

    red = jnp.sum(partials, axis=(0, 2))
    g = red[:c_in * c_in].reshape(c_in, c_in)
    s = red[c_in * c_in:]
    mean = (w_mat @ s) / m
    ey2 = jnp.sum((w_mat @ g) * w_mat, axis=1) / m
    var = jnp.maximum(ey2 - mean * mean, 0.0)
    inv_std = jax.lax.rsqrt(var + _BN_EPS)
    scale = gamma.astype(jnp.float32) * inv_std
    shift = beta.astype(jnp.float32) - mean * scale
    w_scaled = w_mat * scale[:, None]

    hb = 128 if h % 128 == 0 else h
    num_t = h // hb
    out = pl.pallas_call(
        _make_apply_kernel(c_in, c_out),
        out_shape=jax.ShapeDtypeStruct((n, c_out, h, w_sp), x_nchw.dtype),
        grid=(n, num_t),
        in_specs=[
            pl.BlockSpec((1, c_in, hb, w_sp), lambda b, t: (b, 0, t, 0)),
            pl.BlockSpec(memory_space=pltpu.SMEM),
            pl.BlockSpec(memory_space=pltpu.SMEM),
        ],
        out_specs=pl.BlockSpec((1, c_out, hb, w_sp),
                               lambda b, t: (b, 0, t, 0)),
        compiler_params=pltpu.CompilerParams(
            dimension_semantics=("parallel", "parallel")),
        cost_estimate=pl.CostEstimate(
            flops=2 * m * c_in * c_out + m * c_out,
            transcendentals=0,
            bytes_accessed=itemsize * (m * c_in + m * c_out)
            + 4 * (c_in + 1) * c_out),
    )(x_nchw, w_scaled, shift)

    return out


# speedup vs baseline: 12.5725x; 1.2862x over previous
import jax
import jax.numpy as jnp
from jax.experimental import pallas as pl
from jax.experimental.pallas import tpu as pltpu

_BN_EPS = 1e-5


def _make_stats_kernel(c_in, nb):
    def _stats_kernel(x_ref, p_ref):
        sums = {}
        ssum = [None] * c_in
        for b in range(nb):
            xs = [x_ref[b, i] for i in range(c_in)]
            for i in range(c_in):
                for j in range(i, c_in):
                    p = jnp.sum(xs[i] * xs[j], axis=0)
                    sums[(i, j)] = p if b == 0 else sums[(i, j)] + p
                q = jnp.sum(xs[i], axis=0)
                ssum[i] = q if b == 0 else ssum[i] + q
        rows = [sums[(min(i, j), max(i, j))]
                for i in range(c_in) for j in range(c_in)]
        p_ref[0] = jnp.stack(rows + ssum)
    return _stats_kernel


def _make_apply_kernel(c_in, c_out):
    def _apply_kernel(x_ref, w_ref, shift_ref, o_ref):
        xs = [x_ref[0, i] for i in range(c_in)]
        for o in range(c_out):
            acc = xs[0] * w_ref[o, 0] + shift_ref[o]
            for i in range(1, c_in):
                acc += xs[i] * w_ref[o, i]
            o_ref[0, o] = acc
    return _apply_kernel


def kernel(x_nchw, conv_weight, gamma, beta):
    n, c_in, h, w_sp = x_nchw.shape
    c_out = conv_weight.shape[0]
    m = n * h * w_sp
    itemsize = jnp.dtype(x_nchw.dtype).itemsize
    w_mat = conv_weight[:, :, 0, 0].astype(jnp.float32)
    nrows = c_in * c_in + c_in

    nb = 4 if n % 4 == 0 else 1
    g1 = n // nb
    partials = pl.pallas_call(
        _make_stats_kernel(c_in, nb),
        out_shape=jax.ShapeDtypeStruct((g1, nrows, w_sp), jnp.float32),
        grid=(g1,),
        in_specs=[
            pl.BlockSpec((nb, c_in, h, w_sp), lambda i: (i, 0, 0, 0)),
        ],
        out_specs=pl.BlockSpec((1, nrows, w_sp), lambda i: (i, 0, 0)),
        compiler_params=pltpu.CompilerParams(
            dimension_semantics=("parallel",)),
        cost_estimate=pl.CostEstimate(
            flops=2 * m * (c_in * (c_in + 1) // 2 + c_in),
            transcendentals=0,
            bytes_accessed=itemsize * m * c_in + 4 * g1 * nrows * w_sp),
    )(x_nchw)

    red = jnp.sum(partials, axis=(0, 2))
    g = red[:c_in * c_in].reshape(c_in, c_in)
    s = red[c_in * c_in:]
    mean = (w_mat @ s) / m
    ey2 = jnp.sum((w_mat @ g) * w_mat, axis=1) / m
    var = jnp.maximum(ey2 - mean * mean, 0.0)
    inv_std = jax.lax.rsqrt(var + _BN_EPS)
    scale = gamma.astype(jnp.float32) * inv_std
    shift = beta.astype(jnp.float32) - mean * scale
    w_scaled = w_mat * scale[:, None]

    hb = 256 if h % 256 == 0 else (128 if h % 128 == 0 else h)
    num_t = h // hb
    out = pl.pallas_call(
        _make_apply_kernel(c_in, c_out),
        out_shape=jax.ShapeDtypeStruct((n, c_out, h, w_sp), x_nchw.dtype),
        grid=(n, num_t),
        in_specs=[
            pl.BlockSpec((1, c_in, hb, w_sp), lambda b, t: (b, 0, t, 0)),
            pl.BlockSpec(memory_space=pltpu.SMEM),
            pl.BlockSpec(memory_space=pltpu.SMEM),
        ],
        out_specs=pl.BlockSpec((1, c_out, hb, w_sp),
                               lambda b, t: (b, 0, t, 0)),
        compiler_params=pltpu.CompilerParams(
            dimension_semantics=("parallel", "parallel")),
        cost_estimate=pl.CostEstimate(
            flops=2 * m * c_in * c_out + m * c_out,
            transcendentals=0,
            bytes_accessed=itemsize * (m * c_in + m * c_out)
            + 4 * (c_in + 1) * c_out),
    )(x_nchw, w_scaled, shift)

    return out


# speedup vs baseline: 12.7261x; 1.0122x over previous
import jax
import jax.numpy as jnp
from jax.experimental import pallas as pl
from jax.experimental.pallas import tpu as pltpu

_BN_EPS = 1e-5


def _make_stats_kernel(c_in, nb):
    def _stats_kernel(x_ref, p_ref):
        sums = {}
        ssum = [None] * c_in
        for b in range(nb):
            xs = [x_ref[b, i] for i in range(c_in)]
            for i in range(c_in):
                for j in range(i, c_in):
                    p = jnp.sum(xs[i] * xs[j], axis=0)
                    sums[(i, j)] = p if b == 0 else sums[(i, j)] + p
                q = jnp.sum(xs[i], axis=0)
                ssum[i] = q if b == 0 else ssum[i] + q
        rows = [sums[(min(i, j), max(i, j))]
                for i in range(c_in) for j in range(c_in)]
        p_ref[0] = jnp.stack(rows + ssum)
    return _stats_kernel


def _make_apply_kernel(c_in, c_out):
    def _apply_kernel(x_ref, w_ref, shift_ref, o_ref):
        xs = [x_ref[0, i] for i in range(c_in)]
        for o in range(c_out):
            acc = xs[0] * w_ref[o, 0] + shift_ref[o]
            for i in range(1, c_in):
                acc += xs[i] * w_ref[o, i]
            o_ref[0, o] = acc
    return _apply_kernel


def kernel(x_nchw, conv_weight, gamma, beta):
    n, c_in, h, w_sp = x_nchw.shape
    c_out = conv_weight.shape[0]
    m = n * h * w_sp
    itemsize = jnp.dtype(x_nchw.dtype).itemsize
    w_mat = conv_weight[:, :, 0, 0].astype(jnp.float32)
    nrows = c_in * c_in + c_in

    nb = 8 if n % 8 == 0 else (4 if n % 4 == 0 else 1)
    g1 = n // nb
    partials = pl.pallas_call(
        _make_stats_kernel(c_in, nb),
        out_shape=jax.ShapeDtypeStruct((g1, nrows, w_sp), jnp.float32),
        grid=(g1,),
        in_specs=[
            pl.BlockSpec((nb, c_in, h, w_sp), lambda i: (i, 0, 0, 0)),
        ],
        out_specs=pl.BlockSpec((1, nrows, w_sp), lambda i: (i, 0, 0)),
        compiler_params=pltpu.CompilerParams(
            dimension_semantics=("parallel",)),
        cost_estimate=pl.CostEstimate(
            flops=2 * m * (c_in * (c_in + 1) // 2 + c_in),
            transcendentals=0,
            bytes_accessed=itemsize * m * c_in + 4 * g1 * nrows * w_sp),
    )(x_nchw)

    red = jnp.sum(partials, axis=(0, 2))
    g = red[:c_in * c_in].reshape(c_in, c_in)
    s = red[c_in * c_in:]
    mean = (w_mat @ s) / m
    ey2 = jnp.sum((w_mat @ g) * w_mat, axis=1) / m
    var = jnp.maximum(ey2 - mean * mean, 0.0)
    inv_std = jax.lax.rsqrt(var + _BN_EPS)
    scale = gamma.astype(jnp.float32) * inv_std
    shift = beta.astype(jnp.float32) - mean * scale
    w_scaled = w_mat * scale[:, None]

    hb = 256 if h % 256 == 0 else (128 if h % 128 == 0 else h)
    num_t = h // hb
    out = pl.pallas_call(
        _make_apply_kernel(c_in, c_out),
        out_shape=jax.ShapeDtypeStruct((n, c_out, h, w_sp), x_nchw.dtype),
        grid=(n, num_t),
        in_specs=[
            pl.BlockSpec((1, c_in, hb, w_sp), lambda b, t: (b, 0, t, 0)),
            pl.BlockSpec(memory_space=pltpu.SMEM),
            pl.BlockSpec(memory_space=pltpu.SMEM),
        ],
        out_specs=pl.BlockSpec((1, c_out, hb, w_sp),
                               lambda b, t: (b, 0, t, 0)),
        compiler_params=pltpu.CompilerParams(
            dimension_semantics=("parallel", "parallel")),
        cost_estimate=pl.CostEstimate(
            flops=2 * m * c_in * c_out + m * c_out,
            transcendentals=0,
            bytes_accessed=itemsize * (m * c_in + m * c_out)
            + 4 * (c_in + 1) * c_out),
    )(x_nchw, w_scaled, shift)

    return out
